# baseline (device time: 16851 ns/iter reference)
import jax
import jax.numpy as jnp
from jax import lax
from jax.experimental import pallas as pl
from jax.experimental.pallas import tpu as pltpu

R = 64
N_F = 7
N_D = 2
N_X = N_F + N_D
F_ROWS = R * N_F
D_BASE = 2 * F_ROWS


def kernel(x):
    m_per, n = x.shape
    m_glob = 2 * m_per
    n_per = n // 2
    assert 2 * F_ROWS + R * N_D == m_per

    def body(
        x_ref, out_ref,
        stage_src, send_buf, rx_buf, ry_buf, rxf, ryf,
        in_sems, local_sem, sx, rx_sem, sy, ry_sem, srx, sry,
    ):
        mx = lax.axis_index("x")
        my = lax.axis_index("y")
        px = 1 - mx
        py = 1 - my

        barrier_sem = pltpu.get_barrier_semaphore()
        pl.semaphore_signal(
            barrier_sem, inc=1,
            device_id=(px, my), device_id_type=pl.DeviceIdType.MESH,
        )
        pl.semaphore_signal(
            barrier_sem, inc=1,
            device_id=(mx, py), device_id_type=pl.DeviceIdType.MESH,
        )

        def src_row(i):
            if i < N_F:
                return my * F_ROWS + i * R
            return D_BASE + (i - N_F) * R

        def in_copy(i):
            return pltpu.make_async_copy(
                x_ref.at[pl.ds(src_row(i), R), pl.ds(px * n_per, n_per)],
                stage_src.at[i],
                in_sems.at[i],
            )

        for i in range(N_X):
            in_copy(i).start()
        local_copy = pltpu.make_async_copy(
            x_ref.at[:, pl.ds(mx * n_per, n_per)],
            out_ref.at[pl.ds(mx * m_per, m_per), :],
            local_sem.at[0],
        )
        local_copy.start()

        pl.semaphore_wait(barrier_sem, 2)

        def x_rdma(i):
            return pltpu.make_async_remote_copy(
                src_ref=send_buf.at[i],
                dst_ref=rx_buf.at[i],
                send_sem=sx.at[i],
                recv_sem=rx_sem.at[i],
                device_id=(px, my),
                device_id_type=pl.DeviceIdType.MESH,
            )

        def y_rdma(i):
            return pltpu.make_async_remote_copy(
                src_ref=rx_buf.at[i],
                dst_ref=ry_buf.at[i],
                send_sem=sy.at[i],
                recv_sem=ry_sem.at[i],
                device_id=(mx, py),
                device_id_type=pl.DeviceIdType.MESH,
            )

        def rx_store(i):
            row = (
                px * m_per + my * F_ROWS + i * R
                if i < N_F
                else px * m_per + D_BASE + (i - N_F) * R
            )
            return pltpu.make_async_copy(
                rxf.at[i], out_ref.at[pl.ds(row, R), :], srx.at[i]
            )

        def ry_store(i):
            return pltpu.make_async_copy(
                ryf.at[i],
                out_ref.at[pl.ds(px * m_per + py * F_ROWS + i * R, R), :],
                sry.at[i],
            )

        for i in range(N_X):
            in_copy(i).wait()
            send_buf[i, :, :] = stage_src[i].astype(jnp.bfloat16)
            x_rdma(i).start()

        Y_LAG = 3
        y_done = 0

        def drain_y(upto):
            nonlocal y_done
            while y_done < min(upto, N_F):
                i = y_done
                y_rdma(i).wait_recv()
                ryf[i, :, :] = ry_buf[i].astype(jnp.float32)
                ry_store(i).start()
                y_done += 1

        for i in range(N_F):
            x_rdma(i).wait_recv()
            y_rdma(i).start()
            rxf[i, :, :] = rx_buf[i].astype(jnp.float32)
            rx_store(i).start()
            drain_y(i - Y_LAG + 1)

        for j in range(N_D):
            i = N_F + j
            x_rdma(i).wait_recv()
            rxf[i, :, :] = rx_buf[i].astype(jnp.float32)
            rx_store(i).start()
            drain_y(y_done + 1)
        drain_y(N_F)

        local_copy.wait()
        for i in range(N_X):
            x_rdma(i).wait_send()
            rx_store(i).wait()
        for i in range(N_F):
            y_rdma(i).wait_send()
            ry_store(i).wait()

    return pl.pallas_call(
        body,
        out_shape=jax.ShapeDtypeStruct((m_glob, n_per), x.dtype),
        in_specs=[pl.BlockSpec(memory_space=pltpu.MemorySpace.HBM)],
        out_specs=pl.BlockSpec(memory_space=pltpu.MemorySpace.HBM),
        scratch_shapes=[
            pltpu.VMEM((N_X, R, n_per), jnp.float32),
            pltpu.VMEM((N_X, R, n_per), jnp.bfloat16),
            pltpu.VMEM((N_X, R, n_per), jnp.bfloat16),
            pltpu.VMEM((N_F, R, n_per), jnp.bfloat16),
            pltpu.VMEM((N_X, R, n_per), jnp.float32),
            pltpu.VMEM((N_F, R, n_per), jnp.float32),
            pltpu.SemaphoreType.DMA((N_X,)),
            pltpu.SemaphoreType.DMA((1,)),
            pltpu.SemaphoreType.DMA((N_X,)),
            pltpu.SemaphoreType.DMA((N_X,)),
            pltpu.SemaphoreType.DMA((N_F,)),
            pltpu.SemaphoreType.DMA((N_F,)),
            pltpu.SemaphoreType.DMA((N_X,)),
            pltpu.SemaphoreType.DMA((N_F,)),
        ],
        compiler_params=pltpu.CompilerParams(collective_id=0),
    )(x)
